# baseline (speedup 1.0000x reference)
import functools

import jax
import jax.numpy as jnp
from jax import lax
from jax.experimental import pallas as pl
from jax.experimental.pallas import tpu as pltpu
from jax.experimental.pallas import tpu_sc as plsc

N_NODES = 100_000
N_EDGES = 6_400_000
N_PAD = 100_352
TRASH = N_NODES
NW = 32
BLK = 6_400
NBLK = N_EDGES // BLK
VPB = BLK // 16
INIT = N_PAD // 16

_mesh = plsc.VectorSubcoreMesh(core_axis_name="c", subcore_axis_name="s")


@functools.partial(
    pl.kernel,
    out_type=jax.ShapeDtypeStruct((2 * N_PAD,), jnp.float32),
    mesh=_mesh,
    compiler_params=pltpu.CompilerParams(needs_layout_passes=False),
    scratch_types=[
        pltpu.VMEM((BLK,), jnp.int32),
        pltpu.VMEM((BLK,), jnp.int32),
        pltpu.VMEM((BLK,), jnp.int32),
        pltpu.VMEM((VPB // 8, 128), jnp.int32),
        pltpu.VMEM((INIT,), jnp.float32),
        pltpu.VMEM((128,), jnp.float32),
        pltpu.VMEM((16,), jnp.int32),
        pltpu.SemaphoreType.DMA,
        pltpu.SemaphoreType.DMA,
        pltpu.SemaphoreType.DMA,
    ],
)
def _sc_scan(edge_hbm, vtx_hbm, out_hbm, cola_v, colb_v, row_v, idx_v,
             fill_v, zero_v, vtx_v, sema, semb, semr):
    c = lax.axis_index("c")
    s = lax.axis_index("s")
    wid = s * 2 + c

    pltpu.sync_copy(vtx_hbm, vtx_v)
    vv = vtx_v[...]

    zeros16 = jnp.zeros((16,), jnp.float32)
    for z in range(8):
        zero_v[pl.ds(z * 16, 16)] = zeros16
    minf = jnp.full((16,), -jnp.inf, jnp.float32)

    @plsc.parallel_loop(0, INIT // 16, unroll=4)
    def _(i):
        fill_v[pl.ds(i * 16, 16)] = minf

    pltpu.sync_copy(fill_v, out_hbm.at[pl.ds(c * N_PAD + s * INIT, INIT)])
    plsc.subcore_barrier()

    half = c * N_PAD
    ones16 = jnp.full((16,), 0x7FFFFFFF, jnp.int32)

    def start_fetch(g, buf, sem):
        return pltpu.async_copy(edge_hbm.at[1, pl.ds(g * BLK, BLK)], buf, sem)

    def sany(m):
        pc = plsc.all_reduce_population_count(m == 0)
        return pc[0] > 0

    def scan_block(buf):
        @plsc.parallel_loop(0, VPB, step=8, unroll=4,
                            carry=(ones16,) * 8)
        def accs(i, acc):
            base = i * 16
            return tuple(
                jnp.minimum(acc[k], buf[pl.ds(base + 16 * k, 16)] ^ vv)
                for k in range(8)
            )

        m01 = jnp.minimum(accs[0], accs[1])
        m23 = jnp.minimum(accs[2], accs[3])
        m45 = jnp.minimum(accs[4], accs[5])
        m67 = jnp.minimum(accs[6], accs[7])
        return sany(jnp.minimum(jnp.minimum(m01, m23),
                                jnp.minimum(m45, m67)))

    def handle_block(g, buf):
        @pl.when(scan_block(buf))
        def _():
            pltpu.sync_copy(edge_hbm.at[0, pl.ds(g * BLK, BLK)], row_v)

            @plsc.parallel_loop(0, VPB // 8, unroll=2)
            def _(j):
                for t in range(8):
                    i = j * 8 + t
                    cv = buf[pl.ds(i * 16, 16)]
                    rv = row_v[pl.ds(i * 16, 16)]
                    hit = (cv == vv) & (rv != vv)
                    idx_v[j, pl.ds(t * 16, 16)] = jnp.where(
                        hit, rv + half, half + TRASH)

            cps = [
                pltpu.async_copy(zero_v, out_hbm.at[idx_v.at[j]], semr)
                for j in range(VPB // 8)
            ]
            for cp in cps:
                cp.wait()

    start_fetch(wid, cola_v, sema)

    def blk_body(j2, carry):
        ja = 2 * j2
        ga = ja * NW + wid
        gb = ga + NW
        gc = gb + NW

        @pl.when(gb < NBLK)
        def _():
            start_fetch(gb, colb_v, semb)

        pltpu.make_async_copy(edge_hbm.at[1, pl.ds(ga * BLK, BLK)],
                              cola_v, sema).wait()
        handle_block(ga, cola_v)

        @pl.when(gc < NBLK)
        def _():
            start_fetch(gc, cola_v, sema)

        @pl.when(gb < NBLK)
        def _():
            pltpu.make_async_copy(edge_hbm.at[1, pl.ds(gb * BLK, BLK)],
                                  colb_v, semb).wait()
            handle_block(gb, colb_v)

        return carry

    lax.fori_loop(0, NBLK // (2 * NW) + 1, blk_body, 0)


def _merge_body(vtx_ref, x_ref, o_ref):
    m = jnp.maximum(x_ref[0], x_ref[1])
    o_ref[...] = jnp.where(vtx_ref[0] == -1, jnp.float32(0.0), m)


_merge = pl.pallas_call(
    _merge_body,
    out_shape=jax.ShapeDtypeStruct((N_PAD // 128, 128), jnp.float32),
    in_specs=[
        pl.BlockSpec(memory_space=pltpu.SMEM),
        pl.BlockSpec(memory_space=pltpu.VMEM),
    ],
    out_specs=pl.BlockSpec(memory_space=pltpu.VMEM),
)


def kernel(logits, edge_index, vertex):
    del logits
    v = jnp.asarray(vertex, jnp.int32)
    vvec = jnp.full((16,), v, jnp.int32)
    halves = _sc_scan(edge_index, vvec)
    merged = _merge(v.reshape(1), halves.reshape(2, N_PAD // 128, 128))
    return merged.reshape(N_PAD)[:N_NODES].reshape(N_NODES, 1)


# speedup vs baseline: 308.8197x; 308.8197x over previous
import functools

import jax
import jax.numpy as jnp
from jax import lax
from jax.experimental import pallas as pl
from jax.experimental.pallas import tpu as pltpu
from jax.experimental.pallas import tpu_sc as plsc

N_NODES = 100_000
N_EDGES = 6_400_000
N_PAD = 100_352
TRASH = N_NODES
NW = 32
BLK = 6_400
NBLK = N_EDGES // BLK
VPB = BLK // 16
INIT = N_PAD // 16

_mesh = plsc.VectorSubcoreMesh(core_axis_name="c", subcore_axis_name="s")


@functools.partial(
    pl.kernel,
    out_type=jax.ShapeDtypeStruct((2 * N_PAD,), jnp.float32),
    mesh=_mesh,
    compiler_params=pltpu.CompilerParams(needs_layout_passes=False),
    scratch_types=[
        pltpu.VMEM((BLK,), jnp.int32),
        pltpu.VMEM((BLK,), jnp.int32),
        pltpu.VMEM((BLK,), jnp.int32),
        pltpu.VMEM((INIT,), jnp.float32),
        pltpu.VMEM((16,), jnp.float32),
        pltpu.VMEM((16,), jnp.int32),
        pltpu.SemaphoreType.DMA,
        pltpu.SemaphoreType.DMA,
        pltpu.SemaphoreType.DMA,
    ],
)
def _sc_scan(edge_hbm, vtx_hbm, out_hbm, cola_v, colb_v, row_v, fill_v,
             zero_v, vtx_v, sema, semb, semr):
    c = lax.axis_index("c")
    s = lax.axis_index("s")
    wid = s * 2 + c

    pltpu.sync_copy(vtx_hbm, vtx_v)
    vv = vtx_v[...]

    zero_v[...] = jnp.zeros((16,), jnp.float32)
    minf = jnp.full((16,), -jnp.inf, jnp.float32)

    @plsc.parallel_loop(0, INIT // 16, unroll=4)
    def _(i):
        fill_v[pl.ds(i * 16, 16)] = minf

    pltpu.sync_copy(fill_v, out_hbm.at[pl.ds(c * N_PAD + s * INIT, INIT)])
    plsc.subcore_barrier()

    half = c * N_PAD
    ones16 = jnp.full((16,), 0x7FFFFFFF, jnp.int32)

    def start_fetch(g, buf, sem):
        return pltpu.async_copy(edge_hbm.at[1, pl.ds(g * BLK, BLK)], buf, sem)

    def sany(m):
        pc = plsc.all_reduce_population_count(m == 0)
        return pc[0] > 0

    def scan_block(buf):
        @plsc.parallel_loop(0, VPB, step=8, unroll=4,
                            carry=(ones16,) * 8)
        def accs(i, acc):
            base = i * 16
            return tuple(
                jnp.minimum(acc[k], buf[pl.ds(base + 16 * k, 16)] ^ vv)
                for k in range(8)
            )

        m01 = jnp.minimum(accs[0], accs[1])
        m23 = jnp.minimum(accs[2], accs[3])
        m45 = jnp.minimum(accs[4], accs[5])
        m67 = jnp.minimum(accs[6], accs[7])
        return sany(jnp.minimum(jnp.minimum(m01, m23),
                                jnp.minimum(m45, m67)))

    lane = lax.iota(jnp.int32, 16)
    bigv = jnp.full((16,), 0x3FFFFFFF, jnp.int32)

    def first_match_scan(buf, prev):
        @plsc.parallel_loop(0, VPB, step=4, unroll=2, carry=(bigv,) * 4)
        def accs(i, acc):
            out = []
            for k in range(4):
                cv = buf[pl.ds((i + k) * 16, 16)]
                iv = jnp.full((16,), i + k, jnp.int32)
                cand = jnp.where((cv == vv) & (iv > prev), iv, bigv)
                out.append(jnp.minimum(acc[k], cand))
            return tuple(out)

        return jnp.minimum(jnp.minimum(accs[0], accs[1]),
                           jnp.minimum(accs[2], accs[3]))

    def handle_block(g, buf):
        @pl.when(scan_block(buf))
        def _():
            pltpu.sync_copy(edge_hbm.at[0, pl.ds(g * BLK, BLK)], row_v)

            def cond(miv):
                pc = plsc.all_reduce_population_count(miv != bigv)
                return pc[0] > 0

            def body(miv):
                mask = miv != bigv
                flat = jnp.where(mask, miv, 0) * 16 + lane
                rv = plsc.load_gather(row_v, [flat])
                hit = mask & (rv != vv)
                idx = jnp.where(hit, rv + half, half + TRASH + lane)
                pltpu.async_copy(zero_v, out_hbm.at[idx], semr).wait()
                return first_match_scan(buf, miv)

            miv0 = first_match_scan(buf, jnp.full((16,), -1, jnp.int32))
            lax.while_loop(cond, body, miv0)

    start_fetch(wid, cola_v, sema)

    def blk_body(j2, carry):
        ja = 2 * j2
        ga = ja * NW + wid
        gb = ga + NW
        gc = gb + NW

        @pl.when(gb < NBLK)
        def _():
            start_fetch(gb, colb_v, semb)

        pltpu.make_async_copy(edge_hbm.at[1, pl.ds(ga * BLK, BLK)],
                              cola_v, sema).wait()
        handle_block(ga, cola_v)

        @pl.when(gc < NBLK)
        def _():
            start_fetch(gc, cola_v, sema)

        @pl.when(gb < NBLK)
        def _():
            pltpu.make_async_copy(edge_hbm.at[1, pl.ds(gb * BLK, BLK)],
                                  colb_v, semb).wait()
            handle_block(gb, colb_v)

        return carry

    lax.fori_loop(0, NBLK // (2 * NW) + 1, blk_body, 0)


def _merge_body(vtx_ref, x_ref, o_ref):
    m = jnp.maximum(x_ref[0], x_ref[1])
    o_ref[...] = jnp.where(vtx_ref[0] == -1, jnp.float32(0.0), m)


_merge = pl.pallas_call(
    _merge_body,
    out_shape=jax.ShapeDtypeStruct((N_PAD // 128, 128), jnp.float32),
    in_specs=[
        pl.BlockSpec(memory_space=pltpu.SMEM),
        pl.BlockSpec(memory_space=pltpu.VMEM),
    ],
    out_specs=pl.BlockSpec(memory_space=pltpu.VMEM),
)


def kernel(logits, edge_index, vertex):
    del logits
    v = jnp.asarray(vertex, jnp.int32)
    vvec = jnp.full((16,), v, jnp.int32)
    halves = _sc_scan(edge_index, vvec)
    merged = _merge(v.reshape(1), halves.reshape(2, N_PAD // 128, 128))
    return merged.reshape(N_PAD)[:N_NODES].reshape(N_NODES, 1)
